# baseline (speedup 1.0000x reference)
import functools

import jax
import jax.numpy as jnp
from jax import lax
from jax.experimental import pallas as pl
from jax.experimental.pallas import tpu as pltpu
from jax.experimental.pallas import tpu_sc as plsc

N = 10000
E = 320000
IN = 128
H = 256
NREL = 3
B = 64
DH = 2 * H
F32 = jnp.float32

NTILES = 16
EPAD = 327680
EPT = EPAD // NTILES
CH = 128
IB = 2048
NB = EPT // IB
CPB = IB // CH
NDUM = 16
RSTRIDE = 624
RCHUNK = 640
HALF = 128


def _transform(h, W_rel, W_loop, b, apply_relu):
    CIN = h.shape[0]
    BN = 1000
    NBLK = N // BN
    wr = W_rel.reshape(NREL, CIN, 128, 2, HALF).transpose(3, 0, 1, 2, 4)
    wl = W_loop.reshape(CIN, 128, 2, HALF).transpose(2, 0, 1, 3)
    b2 = b.reshape(2, 1, HALF)

    def body(h_ref, wr_ref, wl_ref, b_ref, T_ref, L_ref):
        r = pl.program_id(2)
        hb = h_ref[...]
        if apply_relu:
            hb = jnp.maximum(hb, 0.0)
        wrb = wr_ref[...]
        t = jnp.dot(hb[0], wrb[0, 0, 0], preferred_element_type=F32)
        for k in range(1, CIN):
            t += jnp.dot(hb[k], wrb[0, 0, k], preferred_element_type=F32)
        T_ref[...] = t[None, None]

        @pl.when(r == 0)
        def _():
            wlb = wl_ref[...]
            l = jnp.dot(hb[0], wlb[0, 0], preferred_element_type=F32)
            for k in range(1, CIN):
                l += jnp.dot(hb[k], wlb[0, k], preferred_element_type=F32)
            L_ref[...] = (l + b_ref[0])[None]

    return pl.pallas_call(
        body,
        grid=(NBLK, 2, NREL),
        in_specs=[
            pl.BlockSpec((CIN, BN, 128), lambda i, c, r: (0, i, 0)),
            pl.BlockSpec((1, 1, CIN, 128, HALF), lambda i, c, r: (c, r, 0, 0, 0)),
            pl.BlockSpec((1, CIN, 128, HALF), lambda i, c, r: (c, 0, 0, 0)),
            pl.BlockSpec((1, 1, HALF), lambda i, c, r: (c, 0, 0)),
        ],
        out_specs=[
            pl.BlockSpec((1, 1, BN, HALF), lambda i, c, r: (c, r, i, 0)),
            pl.BlockSpec((1, BN, HALF), lambda i, c, r: (c, i, 0)),
        ],
        out_shape=[
            jax.ShapeDtypeStruct((2, NREL, N, HALF), F32),
            jax.ShapeDtypeStruct((2, N, HALF), F32),
        ],
    )(h, wr, wl, b2)


def _sc_aggregate(T, L, src16, et16, dst16):
    T6 = T.reshape(2 * NREL * N, HALF)
    mesh = plsc.VectorSubcoreMesh(core_axis_name="c", subcore_axis_name="s")

    @functools.partial(
        pl.kernel,
        out_type=jax.ShapeDtypeStruct((2, N, HALF), F32),
        mesh=mesh,
        scratch_types=[
            pltpu.VMEM((2, IB), jnp.int32),
            pltpu.VMEM((2, IB), jnp.int32),
            pltpu.VMEM((2, IB), jnp.int32),
            pltpu.VMEM((CH, HALF), F32),
            pltpu.VMEM((CH, HALF), F32),
            pltpu.VMEM_SHARED((N + NDUM, HALF), F32),
            pltpu.SemaphoreType.DMA,
            pltpu.SemaphoreType.DMA,
            pltpu.SemaphoreType.DMA,
            pltpu.SemaphoreType.DMA,
            pltpu.SemaphoreType.DMA,
            pltpu.SemaphoreType.DMA,
        ],
    )
    def agg(T_hbm, L_hbm, src_hbm, et_hbm, dst_hbm, out_hbm,
            srcb, etb, dstb, buf0, buf1, acc_sh, isem0, isem1, gsem0, gsem1,
            ssem0, ssem1):
        c = lax.axis_index("c")
        s = lax.axis_index("s")
        cbase = c * (NREL * N)
        bufs = (buf0, buf1)
        gsems = (gsem0, gsem1)
        ssems = (ssem0, ssem1)
        isems = (isem0, isem1)

        def icopies(k, p):
            return [pltpu.make_async_copy(hbm.at[s, k], ref.at[p], isems[p])
                    for hbm, ref in ((src_hbm, srcb), (et_hbm, etb),
                                     (dst_hbm, dstb))]

        for cp in icopies(0, 0):
            cp.start()

        pltpu.sync_copy(L_hbm.at[c, pl.ds(s * RSTRIDE, RCHUNK)],
                        acc_sh.at[pl.ds(s * RSTRIDE, RCHUNK)])
        plsc.subcore_barrier()

        for k in range(NB):
            p = k % 2
            for cp in icopies(k, p):
                cp.wait()
            if k + 1 < NB:
                for cp in icopies(k + 1, 1 - p):
                    cp.start()

            def idx_body(i, carry):
                sl = pl.ds(i * 16, 16)
                srcb[p, sl] = etb[p, sl] * N + srcb[p, sl] + cbase
                return carry

            lax.fori_loop(0, IB // 16, idx_body, 0)

            def gcopy(j, bb):
                return pltpu.make_async_copy(
                    T_hbm.at[srcb.at[p].at[pl.ds(j * CH, CH)]],
                    bufs[bb], gsems[bb])

            def scat_start(j, bb):
                pltpu.async_copy(
                    bufs[bb],
                    acc_sh.at[dstb.at[p].at[pl.ds(j * CH, CH)]],
                    ssems[bb], add=True)

            def scat_wait(j, bb):
                pltpu.make_async_copy(
                    bufs[bb],
                    acc_sh.at[dstb.at[p].at[pl.ds(j * CH, CH)]],
                    ssems[bb]).wait()

            gcopy(0, 0).start()

            def chunk_body(i, carry):
                for bb in range(2):
                    j = 2 * i + bb
                    gcopy(j, bb).wait()
                    scat_start(j, bb)

                    @pl.when((j >= 1) & (j + 1 < CPB))
                    def _():
                        scat_wait(j - 1, 1 - bb)

                    @pl.when(j + 1 < CPB)
                    def _():
                        gcopy(j + 1, 1 - bb).start()
                return carry

            lax.fori_loop(0, CPB // 2, chunk_body, 0)
            scat_wait(CPB - 2, 0)
            scat_wait(CPB - 1, 1)

        plsc.subcore_barrier()
        pltpu.sync_copy(acc_sh.at[pl.ds(s * RSTRIDE, RCHUNK)],
                        out_hbm.at[c, pl.ds(s * RSTRIDE, RCHUNK)])

    return agg(T6, L, src16, et16, dst16)


def _pool_mlp(acc3, ids, la, ln, w0a, w0b, w0c, db0, DW1, db1, DW2, db2):
    def body(h_ref, ids_ref, la_ref, ln_ref, w0a_ref, w0b_ref, w0c_ref,
             b0_ref, w1_ref, b1_ref, w2_ref, b2_ref, out_ref):
        sel = (lax.broadcasted_iota(jnp.int32, (B, N), 0)
               == ids_ref[...]).astype(F32)
        h0 = jnp.maximum(h_ref[0], 0.0)
        h1 = jnp.maximum(h_ref[1], 0.0)
        hg0 = jnp.dot(sel, h0, preferred_element_type=F32)
        hg1 = jnp.dot(sel, h1, preferred_element_type=F32)
        w0a = w0a_ref[...]
        z = jnp.dot(hg0, w0a[:HALF], preferred_element_type=F32)
        z += jnp.dot(hg1, w0a[HALF:], preferred_element_type=F32)
        z += la_ref[...] * w0b_ref[...]
        z += jnp.dot(ln_ref[...], w0c_ref[...], preferred_element_type=F32)
        x = jnp.maximum(z + b0_ref[...], 0.0)
        x = jnp.maximum(jnp.dot(x, w1_ref[...], preferred_element_type=F32)
                        + b1_ref[...], 0.0)
        out_ref[...] = (jnp.dot(x, w2_ref[...], preferred_element_type=F32)
                        + b2_ref[...])

    return pl.pallas_call(
        body,
        out_shape=jax.ShapeDtypeStruct((B, 1), F32),
    )(acc3, ids, la, ln, w0a, w0b, w0c, db0, DW1, db1, DW2, db2)


def kernel(node_feats, edge_src, edge_dst, edge_type, node_graph_ids,
           last_action_node, last_node,
           W_rel0, W_loop0, b0, W_rel1, W_loop1, b1, W_rel2, W_loop2, b2,
           DW0, Db0, DW1, Db1, DW2, Db2):
    npad = EPAD - E
    pad0 = jnp.zeros((npad,), jnp.int32)
    src16 = jnp.concatenate([edge_src, pad0]).reshape(NTILES, NB, IB)
    et16 = jnp.concatenate([edge_type, pad0]).reshape(NTILES, NB, IB)
    dst16 = jnp.concatenate(
        [edge_dst, jnp.full((npad,), N, jnp.int32)]).reshape(NTILES, NB, IB)

    h = node_feats.reshape(1, N, IN)
    T, L = _transform(h, W_rel0, W_loop0, b0, apply_relu=False)
    a = _sc_aggregate(T, L, src16, et16, dst16)
    T, L = _transform(a, W_rel1, W_loop1, b1, apply_relu=True)
    a = _sc_aggregate(T, L, src16, et16, dst16)
    T, L = _transform(a, W_rel2, W_loop2, b2, apply_relu=True)
    a = _sc_aggregate(T, L, src16, et16, dst16)

    return _pool_mlp(
        a, node_graph_ids.reshape(1, N), last_action_node, last_node,
        DW0[:H], DW0[H:H + 1], DW0[H + 1:], Db0.reshape(1, DH),
        DW1, Db1.reshape(1, DH), DW2, Db2.reshape(1, 1))


# speedup vs baseline: 1.0892x; 1.0892x over previous
import functools

import jax
import jax.numpy as jnp
from jax import lax
from jax.experimental import pallas as pl
from jax.experimental.pallas import tpu as pltpu
from jax.experimental.pallas import tpu_sc as plsc

N = 10000
E = 320000
IN = 128
H = 256
NREL = 3
B = 64
DH = 2 * H
F32 = jnp.float32

NTILES = 16
EPAD = 327680
EPT = EPAD // NTILES
CH = 64
IB = 2048
NB = EPT // IB
CPB = IB // CH
NBUF = 4
NDUM = 8
RSTRIDE = 624
RCHUNK = 640
HALF = 128


def _transform(h, W_rel, W_loop, b, apply_relu):
    CIN = h.shape[0]
    BN = 1000
    NBLK = N // BN
    wr = W_rel.reshape(NREL, CIN, 128, 2, HALF).transpose(3, 0, 1, 2, 4)
    wl = W_loop.reshape(CIN, 128, 2, HALF).transpose(2, 0, 1, 3)
    b2 = b.reshape(2, 1, HALF)

    def body(h_ref, wr_ref, wl_ref, b_ref, T_ref, L_ref):
        r = pl.program_id(2)
        hb = h_ref[...]
        if apply_relu:
            hb = jnp.maximum(hb, 0.0)
        wrb = wr_ref[...]
        t = jnp.dot(hb[0], wrb[0, 0, 0], preferred_element_type=F32)
        for k in range(1, CIN):
            t += jnp.dot(hb[k], wrb[0, 0, k], preferred_element_type=F32)
        T_ref[...] = t[None, None]

        @pl.when(r == 0)
        def _():
            wlb = wl_ref[...]
            l = jnp.dot(hb[0], wlb[0, 0], preferred_element_type=F32)
            for k in range(1, CIN):
                l += jnp.dot(hb[k], wlb[0, k], preferred_element_type=F32)
            L_ref[...] = (l + b_ref[0])[None]

    return pl.pallas_call(
        body,
        grid=(NBLK, 2, NREL),
        in_specs=[
            pl.BlockSpec((CIN, BN, 128), lambda i, c, r: (0, i, 0)),
            pl.BlockSpec((1, 1, CIN, 128, HALF), lambda i, c, r: (c, r, 0, 0, 0)),
            pl.BlockSpec((1, CIN, 128, HALF), lambda i, c, r: (c, 0, 0, 0)),
            pl.BlockSpec((1, 1, HALF), lambda i, c, r: (c, 0, 0)),
        ],
        out_specs=[
            pl.BlockSpec((1, 1, BN, HALF), lambda i, c, r: (c, r, i, 0)),
            pl.BlockSpec((1, BN, HALF), lambda i, c, r: (c, i, 0)),
        ],
        out_shape=[
            jax.ShapeDtypeStruct((2, NREL, N, HALF), F32),
            jax.ShapeDtypeStruct((2, N, HALF), F32),
        ],
    )(h, wr, wl, b2)


def _sc_aggregate(T, L, src16, et16, dst16):
    T6 = T.reshape(2 * NREL * N, HALF)
    mesh = plsc.VectorSubcoreMesh(core_axis_name="c", subcore_axis_name="s")

    @functools.partial(
        pl.kernel,
        out_type=jax.ShapeDtypeStruct((2, N, HALF), F32),
        mesh=mesh,
        scratch_types=(
            [pltpu.VMEM((IB,), jnp.int32) for _ in range(6)]
            + [pltpu.VMEM((CH, HALF), F32) for _ in range(NBUF)]
            + [pltpu.VMEM_SHARED((N + NDUM, HALF), F32)]
            + [pltpu.SemaphoreType.DMA for _ in range(2 + 2 * NBUF)]
        ),
    )
    def agg(T_hbm, L_hbm, src_hbm, et_hbm, dst_hbm, out_hbm,
            src0, src1, et0, et1, dst0, dst1, b0, b1, b2, b3, acc_sh,
            isem0, isem1, *bsems):
        c = lax.axis_index("c")
        s = lax.axis_index("s")
        cbase = c * (NREL * N)
        srcb = (src0, src1)
        etb = (et0, et1)
        dstb = (dst0, dst1)
        bufs = (b0, b1, b2, b3)
        gsems = bsems[:NBUF]
        ssems = bsems[NBUF:]
        isems = (isem0, isem1)

        def icopies(k, p):
            return [pltpu.make_async_copy(hbm.at[s, k], refs[p], isems[p])
                    for hbm, refs in ((src_hbm, srcb), (et_hbm, etb),
                                     (dst_hbm, dstb))]

        for cp in icopies(0, 0):
            cp.start()

        pltpu.sync_copy(L_hbm.at[c, pl.ds(s * RSTRIDE, RCHUNK)],
                        acc_sh.at[pl.ds(s * RSTRIDE, RCHUNK)])
        plsc.subcore_barrier()

        for k in range(NB):
            p = k % 2
            for cp in icopies(k, p):
                cp.wait()
            if k + 1 < NB:
                for cp in icopies(k + 1, 1 - p):
                    cp.start()

            def idx_body(i, carry):
                for u in range(8):
                    sl = pl.ds(i * 128 + u * 16, 16)
                    srcb[p][sl] = etb[p][sl] * N + srcb[p][sl] + cbase
                return carry

            lax.fori_loop(0, IB // 128, idx_body, 0)

            def gcopy(j, bb):
                return pltpu.make_async_copy(
                    T_hbm.at[srcb[p].at[pl.ds(j * CH, CH)]],
                    bufs[bb], gsems[bb])

            def scat_start(j, bb):
                pltpu.async_copy(
                    bufs[bb],
                    acc_sh.at[dstb[p].at[pl.ds(j * CH, CH)]],
                    ssems[bb], add=True)

            def scat_wait(j, bb):
                pltpu.make_async_copy(
                    bufs[bb],
                    acc_sh.at[dstb[p].at[pl.ds(j * CH, CH)]],
                    ssems[bb]).wait()

            for t in range(NBUF - 1):
                gcopy(t, t).start()

            def chunk_body(i, carry):
                for bb in range(NBUF):
                    j = NBUF * i + bb
                    gcopy(j, bb).wait()
                    scat_start(j, bb)
                    prev = (bb - 1) % NBUF

                    @pl.when((j >= 1) & (j + NBUF - 1 < CPB))
                    def _():
                        scat_wait(j - 1, prev)

                    @pl.when(j + NBUF - 1 < CPB)
                    def _():
                        gcopy(j + NBUF - 1, prev).start()
                return carry

            lax.fori_loop(0, CPB // NBUF, chunk_body, 0)
            for t in range(NBUF):
                scat_wait(CPB - NBUF + t, (CPB - NBUF + t) % NBUF)

        plsc.subcore_barrier()
        pltpu.sync_copy(acc_sh.at[pl.ds(s * RSTRIDE, RCHUNK)],
                        out_hbm.at[c, pl.ds(s * RSTRIDE, RCHUNK)])

    return agg(T6, L, src16, et16, dst16)


def _pool_mlp(acc3, ids, la, ln, w0a, w0b, w0c, db0, DW1, db1, DW2, db2):
    def body(h_ref, ids_ref, la_ref, ln_ref, w0a_ref, w0b_ref, w0c_ref,
             b0_ref, w1_ref, b1_ref, w2_ref, b2_ref, out_ref):
        sel = (lax.broadcasted_iota(jnp.int32, (B, N), 0)
               == ids_ref[...]).astype(F32)
        h0 = jnp.maximum(h_ref[0], 0.0)
        h1 = jnp.maximum(h_ref[1], 0.0)
        hg0 = jnp.dot(sel, h0, preferred_element_type=F32)
        hg1 = jnp.dot(sel, h1, preferred_element_type=F32)
        w0a = w0a_ref[...]
        z = jnp.dot(hg0, w0a[:HALF], preferred_element_type=F32)
        z += jnp.dot(hg1, w0a[HALF:], preferred_element_type=F32)
        z += la_ref[...] * w0b_ref[...]
        z += jnp.dot(ln_ref[...], w0c_ref[...], preferred_element_type=F32)
        x = jnp.maximum(z + b0_ref[...], 0.0)
        x = jnp.maximum(jnp.dot(x, w1_ref[...], preferred_element_type=F32)
                        + b1_ref[...], 0.0)
        out_ref[...] = (jnp.dot(x, w2_ref[...], preferred_element_type=F32)
                        + b2_ref[...])

    return pl.pallas_call(
        body,
        out_shape=jax.ShapeDtypeStruct((B, 1), F32),
    )(acc3, ids, la, ln, w0a, w0b, w0c, db0, DW1, db1, DW2, db2)


def kernel(node_feats, edge_src, edge_dst, edge_type, node_graph_ids,
           last_action_node, last_node,
           W_rel0, W_loop0, b0, W_rel1, W_loop1, b1, W_rel2, W_loop2, b2,
           DW0, Db0, DW1, Db1, DW2, Db2):
    npad = EPAD - E
    pad0 = jnp.zeros((npad,), jnp.int32)
    src16 = jnp.concatenate([edge_src, pad0]).reshape(NTILES, NB, IB)
    et16 = jnp.concatenate([edge_type, pad0]).reshape(NTILES, NB, IB)
    dst16 = jnp.concatenate(
        [edge_dst, jnp.full((npad,), N, jnp.int32)]).reshape(NTILES, NB, IB)

    h = node_feats.reshape(1, N, IN)
    T, L = _transform(h, W_rel0, W_loop0, b0, apply_relu=False)
    a = _sc_aggregate(T, L, src16, et16, dst16)
    T, L = _transform(a, W_rel1, W_loop1, b1, apply_relu=True)
    a = _sc_aggregate(T, L, src16, et16, dst16)
    T, L = _transform(a, W_rel2, W_loop2, b2, apply_relu=True)
    a = _sc_aggregate(T, L, src16, et16, dst16)

    return _pool_mlp(
        a, node_graph_ids.reshape(1, N), last_action_node, last_node,
        DW0[:H], DW0[H:H + 1], DW0[H + 1:], Db0.reshape(1, DH),
        DW1, Db1.reshape(1, DH), DW2, Db2.reshape(1, 1))
